# baseline (speedup 1.0000x reference)
import functools
import jax
import jax.numpy as jnp
from jax import lax
from jax.experimental import pallas as pl
from jax.experimental.pallas import tpu as pltpu
from jax.experimental.pallas import tpu_sc as plsc

_INFO = plsc.get_sparse_core_info()
_NC = _INFO.num_cores
_NS = _INFO.num_subcores
_NW = _NC * _NS

_PITCH = 40


def _relayout(table4d):
    FB, n_blocks, _, _ = table4d.shape
    D = FB * 8
    VP = n_blocks * 128
    iters = (n_blocks + _NW - 1) // _NW

    mesh = plsc.VectorSubcoreMesh(core_axis_name="c", subcore_axis_name="s")

    @functools.partial(
        pl.kernel,
        mesh=mesh,
        out_type=jax.ShapeDtypeStruct((VP, 32), table4d.dtype),
        scratch_types=[
            [pltpu.VMEM((FB, 8, 128), table4d.dtype) for _ in range(2)],
            [pltpu.VMEM((128, _PITCH), table4d.dtype) for _ in range(2)],
            [pltpu.SemaphoreType.DMA for _ in range(2)],
            [pltpu.SemaphoreType.DMA for _ in range(2)],
        ],
        compiler_params=pltpu.CompilerParams(
            use_tc_tiling_on_sc=False, needs_layout_passes=False
        ),
    )
    def k(tbl_hbm, out_hbm, inb, lin, g_sems, w_sems):
        wid = lax.axis_index("s") * _NC + lax.axis_index("c")
        j = lax.iota(jnp.int32, 16)

        def blk_id(i):
            return wid + i * _NW

        def in_copy(i, p):
            return pltpu.make_async_copy(
                tbl_hbm.at[:, blk_id(i)], inb[p], g_sems[p]
            )

        def start_in(i, p):
            in_copy(i, p).start()

        def wait_in(i, p):
            in_copy(i, p).wait()

        def out_copy(i, p):
            return pltpu.make_async_copy(
                lin[p].at[:, pl.ds(0, 32)],
                out_hbm.at[pl.ds(blk_id(i) * 128, 128), :],
                w_sems[p],
            )

        jrk = [j + kk * 16 for kk in range(8)]

        def transpose(p):
            for f in range(D):
                fb, fi = divmod(f, 8)
                fv = jnp.broadcast_to(jnp.int32(f), (16,))
                for kk in range(8):
                    vals = inb[p][fb, fi, pl.ds(kk * 16, 16)]
                    plsc.store_scatter(lin[p], [jrk[kk], fv], vals)

        def do_iter(i, p):
            wait_in(i, p)

            @pl.when(i >= 2)
            def _():
                out_copy(i - 2, p).wait()

            transpose(p)

            @pl.when(blk_id(i + 2) < n_blocks)
            def _():
                start_in(i + 2, p)

            out_copy(i, p).start()

        @pl.when(blk_id(0) < n_blocks)
        def _():
            start_in(0, 0)

        @pl.when(blk_id(1) < n_blocks)
        def _():
            start_in(1, 1)

        def it_loop(i, _):
            @pl.when((i % 2 == 0) & (blk_id(i) < n_blocks))
            def _():
                do_iter(i, 0)

            @pl.when((i % 2 == 1) & (blk_id(i) < n_blocks))
            def _():
                do_iter(i, 1)

            return 0

        lax.fori_loop(0, iters, it_loop, 0)

        @pl.when(blk_id(iters - 2) < n_blocks)
        def _():
            out_copy(iters - 2, (iters - 2) % 2).wait()

        @pl.when(blk_id(iters - 1) < n_blocks)
        def _():
            out_copy(iters - 1, (iters - 1) % 2).wait()

    return k(table4d)


def _gather(scratch, index, B):
    b_per_w = B // _NW
    C = 512
    BLK = C // 128
    n_chunks = b_per_w // C
    blocks_per_w = b_per_w // 128
    TOTB = B // 128

    scratch2d = scratch

    mesh = plsc.VectorSubcoreMesh(core_axis_name="c", subcore_axis_name="s")

    @functools.partial(
        pl.kernel,
        mesh=mesh,
        out_type=jax.ShapeDtypeStruct((4, TOTB, 8, 128), jnp.float32),
        scratch_types=[
            pltpu.VMEM((b_per_w,), jnp.int32),
            [pltpu.VMEM((C, 32), jnp.float32) for _ in range(2)],
            [pltpu.VMEM((BLK, 32, 129), jnp.float32) for _ in range(2)],
            [pltpu.SemaphoreType.DMA for _ in range(2)],
            [pltpu.SemaphoreType.DMA for _ in range(2)],
        ],
        compiler_params=pltpu.CompilerParams(
            use_tc_tiling_on_sc=False, needs_layout_passes=False
        ),
    )
    def k(tbl_hbm, idx_hbm, out_hbm, idx_v, rows, nat, g_sems, w_sems):
        wid = lax.axis_index("s") * _NC + lax.axis_index("c")
        base = wid * b_per_w
        blk0 = wid * blocks_per_w
        pltpu.sync_copy(idx_hbm.at[pl.ds(base, b_per_w)], idx_v)

        j = lax.iota(jnp.int32, 16)
        jf = [j + h * 16 for h in range(2)]

        def start_gather(c, p):
            pltpu.async_copy(
                tbl_hbm.at[idx_v.at[pl.ds(c * C, C)]], rows[p], g_sems[p]
            )

        def wait_gather(c, p):
            pltpu.make_async_copy(
                tbl_hbm.at[idx_v.at[pl.ds(c * C, C)]], rows[p], g_sems[p]
            ).wait()

        def out_copy(c, p, fb, blk):
            return pltpu.make_async_copy(
                nat[p].at[blk, pl.ds(fb * 8, 8), pl.ds(0, 128)],
                out_hbm.at[fb, blk0 + c * BLK + blk],
                w_sems[p],
            )

        def start_write(c, p):
            for blk in range(BLK):
                for fb in range(4):
                    out_copy(c, p, fb, blk).start()

        def wait_write(c, p):
            for blk in range(BLK):
                for fb in range(4):
                    out_copy(c, p, fb, blk).wait()

        def transpose(p):
            def blk_body(blk, _):
                rowbase = blk * 128
                blkv = jnp.broadcast_to(blk, (16,))

                def r_body(r2, _):
                    for dr in range(2):
                        r = r2 * 2 + dr
                        rv = jnp.broadcast_to(r, (16,))
                        for h in range(2):
                            vals = rows[p][rowbase + r, pl.ds(h * 16, 16)]
                            plsc.store_scatter(nat[p], [blkv, jf[h], rv], vals)
                    return 0

                lax.fori_loop(0, 64, r_body, 0)
                return 0

            lax.fori_loop(0, BLK, blk_body, 0)

        def do_chunk(c, p):
            wait_gather(c, p)

            @pl.when(c >= 2)
            def _():
                wait_write(c - 2, p)

            transpose(p)

            @pl.when(c + 2 < n_chunks)
            def _():
                start_gather(c + 2, p)

            start_write(c, p)

        start_gather(0, 0)
        start_gather(1, 1)

        def chunk_loop(c, _):
            @pl.when(c % 2 == 0)
            def _():
                do_chunk(c, 0)

            @pl.when(c % 2 == 1)
            def _():
                do_chunk(c, 1)

            return 0

        lax.fori_loop(0, n_chunks, chunk_loop, 0)
        wait_write(n_chunks - 2, 0)
        wait_write(n_chunks - 1, 1)

    return k(scratch2d, index)


def kernel(input, index, _):
    B, = index.shape
    V, D = input.shape
    VP = ((V + 127) // 128) * 128
    padded = jnp.pad(input, ((0, VP - V), (0, 0)))
    table4d = (
        padded.T.reshape(D // 8, 8, VP // 128, 128).transpose(0, 2, 1, 3)
    )
    scratch = _relayout(table4d)
    out4d = _gather(scratch, index, B)
    out = out4d.transpose(0, 2, 1, 3).reshape(D, B).T
    return (input, index, out)


# speedup vs baseline: 1.2760x; 1.2760x over previous
import functools
import jax
import jax.numpy as jnp
from jax import lax
from jax.experimental import pallas as pl
from jax.experimental.pallas import tpu as pltpu
from jax.experimental.pallas import tpu_sc as plsc

_INFO = plsc.get_sparse_core_info()
_NC = _INFO.num_cores
_NS = _INFO.num_subcores
_NW = _NC * _NS

_PITCH = 40


def _relayout(table4d):
    FB, n_blocks, _, _ = table4d.shape
    D = FB * 8
    VP = n_blocks * 128
    iters = (n_blocks + _NW - 1) // _NW

    mesh = plsc.VectorSubcoreMesh(core_axis_name="c", subcore_axis_name="s")

    @functools.partial(
        pl.kernel,
        mesh=mesh,
        out_type=jax.ShapeDtypeStruct((VP * _PITCH,), table4d.dtype),
        scratch_types=[
            [pltpu.VMEM((FB, 8, 128), table4d.dtype) for _ in range(2)],
            [pltpu.VMEM((128 * _PITCH,), table4d.dtype) for _ in range(2)],
            [pltpu.SemaphoreType.DMA for _ in range(2)],
            [pltpu.SemaphoreType.DMA for _ in range(2)],
        ],
        compiler_params=pltpu.CompilerParams(
            use_tc_tiling_on_sc=False, needs_layout_passes=False
        ),
    )
    def k(tbl_hbm, out_hbm, inb, lin, g_sems, w_sems):
        wid = lax.axis_index("s") * _NC + lax.axis_index("c")
        j = lax.iota(jnp.int32, 16)
        jp = j * _PITCH

        def blk_id(i):
            return wid + i * _NW

        def in_copy(i, p):
            return pltpu.make_async_copy(
                tbl_hbm.at[:, blk_id(i)], inb[p], g_sems[p]
            )

        def start_in(i, p):
            in_copy(i, p).start()

        def wait_in(i, p):
            in_copy(i, p).wait()

        def out_copy(i, p):
            return pltpu.make_async_copy(
                lin[p],
                out_hbm.at[pl.ds(blk_id(i) * 128 * _PITCH, 128 * _PITCH)],
                w_sems[p],
            )

        jpk = [jp + kk * 16 * _PITCH for kk in range(8)]

        def transpose(p):
            for f in range(D):
                fb, fi = divmod(f, 8)
                for kk in range(8):
                    vals = inb[p][fb, fi, pl.ds(kk * 16, 16)]
                    plsc.store_scatter(lin[p], [jpk[kk] + f], vals)

        def do_iter(i, p):
            wait_in(i, p)

            @pl.when(i >= 2)
            def _():
                out_copy(i - 2, p).wait()

            transpose(p)

            @pl.when(blk_id(i + 2) < n_blocks)
            def _():
                start_in(i + 2, p)

            out_copy(i, p).start()

        @pl.when(blk_id(0) < n_blocks)
        def _():
            start_in(0, 0)

        @pl.when(blk_id(1) < n_blocks)
        def _():
            start_in(1, 1)

        def it_loop(i, _):
            @pl.when((i % 2 == 0) & (blk_id(i) < n_blocks))
            def _():
                do_iter(i, 0)

            @pl.when((i % 2 == 1) & (blk_id(i) < n_blocks))
            def _():
                do_iter(i, 1)

            return 0

        lax.fori_loop(0, iters, it_loop, 0)

        @pl.when(blk_id(iters - 2) < n_blocks)
        def _():
            out_copy(iters - 2, (iters - 2) % 2).wait()

        @pl.when(blk_id(iters - 1) < n_blocks)
        def _():
            out_copy(iters - 1, (iters - 1) % 2).wait()

    return k(table4d)


def _gather(scratch, index, B):
    b_per_w = B // _NW
    C = 512
    BLK = C // 128
    n_chunks = b_per_w // C
    blocks_per_w = b_per_w // 128
    TOTB = B // 128

    scratch2d = scratch.reshape(-1, _PITCH)

    mesh = plsc.VectorSubcoreMesh(core_axis_name="c", subcore_axis_name="s")

    @functools.partial(
        pl.kernel,
        mesh=mesh,
        out_type=jax.ShapeDtypeStruct((4, TOTB, 8, 128), jnp.float32),
        scratch_types=[
            pltpu.VMEM((b_per_w,), jnp.int32),
            [pltpu.VMEM((C, _PITCH), jnp.float32) for _ in range(2)],
            [pltpu.VMEM((BLK, 32, 129), jnp.float32) for _ in range(2)],
            [pltpu.SemaphoreType.DMA for _ in range(2)],
            [pltpu.SemaphoreType.DMA for _ in range(2)],
        ],
        compiler_params=pltpu.CompilerParams(
            use_tc_tiling_on_sc=False, needs_layout_passes=False
        ),
    )
    def k(tbl_hbm, idx_hbm, out_hbm, idx_v, rows, nat, g_sems, w_sems):
        wid = lax.axis_index("s") * _NC + lax.axis_index("c")
        base = wid * b_per_w
        blk0 = wid * blocks_per_w
        pltpu.sync_copy(idx_hbm.at[pl.ds(base, b_per_w)], idx_v)

        j = lax.iota(jnp.int32, 16)
        jf = [j + h * 16 for h in range(2)]

        def start_gather(c, p):
            pltpu.async_copy(
                tbl_hbm.at[idx_v.at[pl.ds(c * C, C)]], rows[p], g_sems[p]
            )

        def wait_gather(c, p):
            pltpu.make_async_copy(
                tbl_hbm.at[idx_v.at[pl.ds(c * C, C)]], rows[p], g_sems[p]
            ).wait()

        def out_copy(c, p, fb, blk):
            return pltpu.make_async_copy(
                nat[p].at[blk, pl.ds(fb * 8, 8), pl.ds(0, 128)],
                out_hbm.at[fb, blk0 + c * BLK + blk],
                w_sems[p],
            )

        def start_write(c, p):
            for blk in range(BLK):
                for fb in range(4):
                    out_copy(c, p, fb, blk).start()

        def wait_write(c, p):
            for blk in range(BLK):
                for fb in range(4):
                    out_copy(c, p, fb, blk).wait()

        def transpose(p):
            def blk_body(blk, _):
                rowbase = blk * 128
                blkv = jnp.broadcast_to(blk, (16,))

                def r_body(r2, _):
                    for dr in range(2):
                        r = r2 * 2 + dr
                        rv = jnp.broadcast_to(r, (16,))
                        for h in range(2):
                            vals = rows[p][rowbase + r, pl.ds(h * 16, 16)]
                            plsc.store_scatter(nat[p], [blkv, jf[h], rv], vals)
                    return 0

                lax.fori_loop(0, 64, r_body, 0)
                return 0

            lax.fori_loop(0, BLK, blk_body, 0)

        def do_chunk(c, p):
            wait_gather(c, p)

            @pl.when(c >= 2)
            def _():
                wait_write(c - 2, p)

            transpose(p)

            @pl.when(c + 2 < n_chunks)
            def _():
                start_gather(c + 2, p)

            start_write(c, p)

        start_gather(0, 0)
        start_gather(1, 1)

        def chunk_loop(c, _):
            @pl.when(c % 2 == 0)
            def _():
                do_chunk(c, 0)

            @pl.when(c % 2 == 1)
            def _():
                do_chunk(c, 1)

            return 0

        lax.fori_loop(0, n_chunks, chunk_loop, 0)
        wait_write(n_chunks - 2, 0)
        wait_write(n_chunks - 1, 1)

    return k(scratch2d, index)


def kernel(input, index, _):
    B, = index.shape
    V, D = input.shape
    VP = ((V + 127) // 128) * 128
    padded = jnp.pad(input, ((0, VP - V), (0, 0)))
    table4d = (
        padded.T.reshape(D // 8, 8, VP // 128, 128).transpose(0, 2, 1, 3)
    )
    scratch = _relayout(table4d)
    out4d = _gather(scratch, index, B)
    out = out4d.transpose(0, 2, 1, 3).reshape(D, B).T
    return (input, index, out)
